# baseline (speedup 1.0000x reference)
import functools

import jax
import jax.numpy as jnp
from jax import lax
from jax.experimental import pallas as pl
from jax.experimental.pallas import tpu as pltpu
from jax.experimental.pallas import tpu_sc as plsc

_NC, _NS = 2, 16
_NW = _NC * _NS
_BATCH = 4
_SEQ = 1024
_B = 4096
_D = 2048
_RPW = _B // _NW
_CHUNK = 8
_NBUF = 6
_NCHUNK = _RPW // _CHUNK

_mesh = plsc.VectorSubcoreMesh(core_axis_name="c", subcore_axis_name="s")


@functools.partial(
    pl.kernel,
    mesh=_mesh,
    out_type=jax.ShapeDtypeStruct((_B, _D), jnp.float32),
    scratch_types=[
        pltpu.VMEM((_RPW,), jnp.int32),
        pltpu.VMEM((_NBUF * _CHUNK, _D), jnp.float32),
        pltpu.SemaphoreType.DMA((_NBUF,)),
        pltpu.SemaphoreType.DMA((_NBUF,)),
    ],
)
def _sc_gather(idx_hbm, table_hbm, out_hbm, idx_v, rows_v, gsem, wsem):
    wid = lax.axis_index("s") * _NC + lax.axis_index("c")
    base = wid * _RPW
    pltpu.sync_copy(
        idx_hbm.at[wid // (_SEQ // _RPW), pl.ds((wid % (_SEQ // _RPW)) * _RPW, _RPW)],
        idx_v,
    )

    gathers = [None] * _NCHUNK
    writes = [None] * _NCHUNK

    def start_gather(g):
        b = g % _NBUF
        gathers[g] = pltpu.async_copy(
            table_hbm.at[idx_v.at[pl.ds(g * _CHUNK, _CHUNK)]],
            rows_v.at[pl.ds(b * _CHUNK, _CHUNK)],
            gsem.at[b],
        )

    npair = _NCHUNK // 2
    pbuf = _NBUF // 2
    for p in range(pbuf):
        start_gather(2 * p)
        start_gather(2 * p + 1)

    for p in range(npair):
        b = (p % pbuf) * 2
        gathers[2 * p].wait()
        gathers[2 * p + 1].wait()
        writes[p] = pltpu.async_copy(
            rows_v.at[pl.ds(b * _CHUNK, 2 * _CHUNK)],
            out_hbm.at[pl.ds(base + 2 * p * _CHUNK, 2 * _CHUNK)],
            wsem.at[b],
        )
        prev = p - 1
        if prev >= 0 and prev + pbuf < npair:
            writes[prev].wait()
            start_gather(2 * (prev + pbuf))
            start_gather(2 * (prev + pbuf) + 1)

    for p in range(npair - pbuf, npair):
        writes[p].wait()


def kernel(indices, table):
    out = _sc_gather(indices.astype(jnp.int32), table)
    return out.reshape(indices.shape + (table.shape[1],))


# speedup vs baseline: 1.0129x; 1.0129x over previous
import functools

import jax
import jax.numpy as jnp
from jax import lax
from jax.experimental import pallas as pl
from jax.experimental.pallas import tpu as pltpu
from jax.experimental.pallas import tpu_sc as plsc

_NC, _NS = 2, 16
_NW = _NC * _NS
_BATCH = 4
_SEQ = 1024
_B = 4096
_D = 2048
_RPW = _B // _NW
_CHUNK = 8
_NBUF = 6
_NCHUNK = _RPW // _CHUNK

_mesh = plsc.VectorSubcoreMesh(core_axis_name="c", subcore_axis_name="s")


@functools.partial(
    pl.kernel,
    mesh=_mesh,
    out_type=jax.ShapeDtypeStruct((_B, _D), jnp.float32),
    scratch_types=[
        pltpu.VMEM((_RPW,), jnp.int32),
        pltpu.VMEM((_NBUF, _CHUNK, _D), jnp.float32),
        pltpu.SemaphoreType.DMA((_NBUF,)),
        pltpu.SemaphoreType.DMA((_NBUF,)),
    ],
)
def _sc_gather(idx_hbm, table_hbm, out_hbm, idx_v, rows_v, gsem, wsem):
    wid = lax.axis_index("s") * _NC + lax.axis_index("c")
    base = wid * _RPW
    pltpu.sync_copy(
        idx_hbm.at[wid // (_SEQ // _RPW), pl.ds((wid % (_SEQ // _RPW)) * _RPW, _RPW)],
        idx_v,
    )

    gathers = [None] * _NCHUNK
    writes = [None] * _NCHUNK

    def start_gather(g):
        b = g % _NBUF
        gathers[g] = pltpu.async_copy(
            table_hbm.at[idx_v.at[pl.ds(g * _CHUNK, _CHUNK)]],
            rows_v.at[b],
            gsem.at[b],
        )

    for g in range(_NBUF):
        start_gather(g)

    for g in range(_NCHUNK):
        b = g % _NBUF
        gathers[g].wait()
        writes[g] = pltpu.async_copy(
            rows_v.at[b],
            out_hbm.at[pl.ds(base + g * _CHUNK, _CHUNK)],
            wsem.at[b],
        )
        prev = g - 1
        if prev >= 0 and prev + _NBUF < _NCHUNK:
            writes[prev].wait()
            start_gather(prev + _NBUF)

    for g in range(_NCHUNK - _NBUF, _NCHUNK):
        writes[g].wait()


def kernel(indices, table):
    out = _sc_gather(indices.astype(jnp.int32), table)
    return out.reshape(indices.shape + (table.shape[1],))
